# baseline (speedup 1.0000x reference)
import functools

import jax
import jax.numpy as jnp
from jax import lax
from jax.experimental import pallas as pl
from jax.experimental.pallas import tpu as pltpu
from jax.experimental.pallas import tpu_sc as plsc

N = 10000
D = 128
E = 320000
NPAD = 10240
NC, NS, L = 2, 16, 16
NW = NC * NS
CH = 128
CPW = 80
CPB = 8
NB = CPW // CPB
E_PAD = NW * CPW * CH
CPW0 = 160
CPW1 = 2 * CPW - CPW0
NB0, NB1 = CPW0 // CPB, CPW1 // CPB
ROWS_PT = NPAD // NS
BM = 1280

_mesh = plsc.VectorSubcoreMesh(core_axis_name="c", subcore_axis_name="s")
_mesh1 = plsc.VectorSubcoreMesh(core_axis_name="c", subcore_axis_name="s",
                                num_cores=1)
CPW1C = E_PAD // CH // NS
NB1C = CPW1C // CPB


@functools.partial(
    pl.kernel,
    out_type=jax.ShapeDtypeStruct((NC * NPAD, D), jnp.float32),
    mesh=_mesh,
    scratch_types=[
        pltpu.VMEM((CPB, CH), jnp.int32),
        pltpu.VMEM((CH, D), jnp.float32),
        pltpu.SemaphoreType.DMA,
        pltpu.VMEM_SHARED((NPAD, D), jnp.float32),
    ],
)
def _deg_kernel(dst_hbm, out_hbm, idx_v, ones_v, sem, hist_s):
    c = lax.axis_index("c")
    s = lax.axis_index("s")
    wid = c * NS + s
    zero = jnp.zeros((L,), jnp.float32)
    one = jnp.ones((L,), jnp.float32)

    def fill(val):
        def frow(i, _):
            def fcol(j, _):
                ones_v[i, pl.ds(j * L, L)] = val
                return 0

            lax.fori_loop(0, D // L, fcol, 0)
            return 0

        lax.fori_loop(0, CH, frow, 0)

    fill(zero)

    def zacc(k, _):
        pltpu.sync_copy(ones_v, hist_s.at[pl.ds(s * ROWS_PT + k * CH, CH)])
        return 0

    lax.fori_loop(0, ROWS_PT // CH, zacc, 0)
    fill(one)
    plsc.subcore_barrier()

    def body(blk, _):
        base_chunk = wid * CPW + blk * CPB
        pltpu.sync_copy(dst_hbm.at[pl.ds(base_chunk, CPB)], idx_v)
        descs = [
            pltpu.async_copy(ones_v, hist_s.at[idx_v.at[p]], sem, add=True)
            for p in range(CPB)
        ]
        for d in descs:
            d.wait()
        return 0

    lax.fori_loop(0, NB, body, 0)
    plsc.subcore_barrier()

    def wout(k, _):
        base = s * ROWS_PT + k * CH
        pltpu.sync_copy(hist_s.at[pl.ds(base, CH)], ones_v)
        pltpu.sync_copy(ones_v, out_hbm.at[pl.ds(c * NPAD + base, CH)])
        return 0

    lax.fori_loop(0, ROWS_PT // CH, wout, 0)


@functools.partial(
    pl.kernel,
    out_type=jax.ShapeDtypeStruct((NC * NPAD, D), jnp.float32),
    mesh=_mesh,
    scratch_types=[
        pltpu.VMEM((CPB, CH), jnp.int32),
        pltpu.VMEM((CPB, CH), jnp.int32),
        pltpu.VMEM((CH, D), jnp.float32),
        pltpu.VMEM((CH, D), jnp.float32),
        pltpu.VMEM_SHARED((NPAD, D), jnp.float32),
        pltpu.SemaphoreType.DMA,
        pltpu.SemaphoreType.DMA,
    ],
)
def _scatter_kernel(y_hbm, src_hbm, dst_hbm, out_hbm, sidx_v, didx_v,
                    rows0_v, rows1_v, acc_s, sem0, sem1):
    c = lax.axis_index("c")
    s = lax.axis_index("s")
    wid = c * NS + s
    zero = jnp.zeros((L,), jnp.float32)

    def zrow(i, _):
        def zcol(j, _):
            rows0_v[i, pl.ds(j * L, L)] = zero
            return 0

        lax.fori_loop(0, D // L, zcol, 0)
        return 0

    lax.fori_loop(0, CH, zrow, 0)

    def zacc(k, _):
        pltpu.sync_copy(rows0_v, acc_s.at[pl.ds(s * ROWS_PT + k * CH, CH)])
        return 0

    lax.fori_loop(0, ROWS_PT // CH, zacc, 0)
    plsc.subcore_barrier()

    rows = (rows0_v, rows1_v)
    sems = (sem0, sem1)
    base_w = jnp.where(c == 0, s * CPW0, NS * CPW0 + s * CPW1)
    nb = jnp.where(c == 0, NB0, NB1)

    def body(blk, _):
        base_chunk = base_w + blk * CPB
        pltpu.sync_copy(src_hbm.at[pl.ds(base_chunk, CPB)], sidx_v)
        pltpu.sync_copy(dst_hbm.at[pl.ds(base_chunk, CPB)], didx_v)
        descs = [
            pltpu.async_copy(y_hbm.at[sidx_v.at[p]], rows[p % 2],
                             sems[p % 2])
            for p in range(2)
        ]
        for p in range(CPB):
            descs[p].wait()
            pltpu.sync_copy(rows[p % 2], acc_s.at[didx_v.at[p]], add=True)
            if p + 2 < CPB:
                descs.append(
                    pltpu.async_copy(y_hbm.at[sidx_v.at[p + 2]],
                                     rows[p % 2], sems[p % 2]))
        return 0

    lax.fori_loop(0, nb, body, 0)
    plsc.subcore_barrier()

    wdescs = {}
    for k in range(ROWS_PT // CH):
        b = k % 2
        base = s * ROWS_PT + k * CH
        if k >= 2:
            wdescs[k - 2].wait()
        pltpu.sync_copy(acc_s.at[pl.ds(base, CH)], rows[b])
        wdescs[k] = pltpu.async_copy(
            rows[b], out_hbm.at[pl.ds(c * NPAD + base, CH)], sems[b])
    for k in range(ROWS_PT // CH - 2, ROWS_PT // CH):
        wdescs[k].wait()


def _dis_of(h_ref):
    deg = h_ref[0, :, 0] + h_ref[1, :, 0] + 1.0
    return lax.rsqrt(deg)[:, None]


def _tc_first(x_ref, w_ref, h_ref, y_ref):
    xw = jnp.dot(x_ref[...], w_ref[...], preferred_element_type=jnp.float32)
    y_ref[...] = xw * _dis_of(h_ref)


def _tc_mid(acc_ref, y_ref, h_ref, w_ref, b_ref, out_ref):
    dis = _dis_of(h_ref)
    tmp = (acc_ref[0] + acc_ref[1] + y_ref[...]) * dis + b_ref[...]
    hcur = jnp.maximum(tmp, 0.0)
    out_ref[...] = (
        jnp.dot(hcur, w_ref[...], preferred_element_type=jnp.float32) * dis
    )


def _tc_last(acc_ref, y_ref, h_ref, b_ref, out_ref):
    dis = _dis_of(h_ref)
    out_ref[...] = (acc_ref[0] + acc_ref[1] + y_ref[...]) * dis + b_ref[...]


_row_spec = pl.BlockSpec((BM, D), lambda i: (i, 0))
_acc_spec = pl.BlockSpec((NC, BM, D), lambda i: (0, i, 0))
_hist_spec = pl.BlockSpec((NC, BM, D), lambda i: (0, i, 0))
_w_spec = pl.BlockSpec((D, D), lambda i: (0, 0))
_b_spec = pl.BlockSpec((1, D), lambda i: (0, 0))
_out_struct = jax.ShapeDtypeStruct((NPAD, D), jnp.float32)

_tc_first_call = pl.pallas_call(
    _tc_first,
    grid=(NPAD // BM,),
    in_specs=[_row_spec, _w_spec, _hist_spec],
    out_specs=_row_spec,
    out_shape=_out_struct,
)

_tc_mid_call = pl.pallas_call(
    _tc_mid,
    grid=(NPAD // BM,),
    in_specs=[_acc_spec, _row_spec, _hist_spec, _w_spec, _b_spec],
    out_specs=_row_spec,
    out_shape=_out_struct,
)

_tc_last_call = pl.pallas_call(
    _tc_last,
    grid=(NPAD // BM,),
    in_specs=[_acc_spec, _row_spec, _hist_spec, _b_spec],
    out_specs=_row_spec,
    out_shape=_out_struct,
)


def kernel(x, edge_index, W1, b1, W2, b2):
    ei = edge_index.astype(jnp.int32)
    npad_e = E_PAD - E
    src = jnp.concatenate([ei[0], jnp.zeros((npad_e,), jnp.int32)])
    pad_dst = N + jnp.arange(npad_e, dtype=jnp.int32) % (NPAD - N)
    dst = jnp.concatenate([ei[1], pad_dst])
    src2 = src.reshape(E_PAD // CH, CH)
    dst2 = dst.reshape(E_PAD // CH, CH)
    x_p = jnp.pad(x, ((0, NPAD - N), (0, 0)))
    b1r = b1.reshape(1, D)
    b2r = b2.reshape(1, D)

    hist = _deg_kernel(dst2).reshape(NC, NPAD, D)
    y1 = _tc_first_call(x_p, W1, hist)
    acc1 = _scatter_kernel(y1, src2, dst2).reshape(NC, NPAD, D)
    y2 = _tc_mid_call(acc1, y1, hist, W2, b1r)
    acc2 = _scatter_kernel(y2, src2, dst2).reshape(NC, NPAD, D)
    out = _tc_last_call(acc2, y2, hist, b2r)
    return out[:N]


# speedup vs baseline: 1.2934x; 1.2934x over previous
import functools

import jax
import jax.numpy as jnp
from jax import lax
from jax.experimental import pallas as pl
from jax.experimental.pallas import tpu as pltpu
from jax.experimental.pallas import tpu_sc as plsc

N = 10000
D = 128
E = 320000
NPAD = 10240
NC, NS, L = 2, 16, 16
NW = NC * NS
CH = 128
CPW = 80
CPB = 8
NB = CPW // CPB
E_PAD = NW * CPW * CH
CPW0 = 128
CPW1 = 2 * CPW - CPW0
NB0, NB1 = CPW0 // CPB, CPW1 // CPB
ROWS_PT = NPAD // NS
BM = 1280

_mesh = plsc.VectorSubcoreMesh(core_axis_name="c", subcore_axis_name="s")
_mesh1 = plsc.VectorSubcoreMesh(core_axis_name="c", subcore_axis_name="s",
                                num_cores=1)
CPW1C = E_PAD // CH // NS
NB1C = CPW1C // CPB


@functools.partial(
    pl.kernel,
    out_type=jax.ShapeDtypeStruct((NC * NPAD, D), jnp.float32),
    mesh=_mesh,
    scratch_types=[
        pltpu.VMEM((CPB, CH), jnp.int32),
        pltpu.VMEM((CH, D), jnp.float32),
        pltpu.SemaphoreType.DMA,
        pltpu.VMEM_SHARED((NPAD, D), jnp.float32),
    ],
)
def _deg_kernel(dst_hbm, out_hbm, idx_v, ones_v, sem, hist_s):
    c = lax.axis_index("c")
    s = lax.axis_index("s")
    wid = c * NS + s
    zero = jnp.zeros((L,), jnp.float32)
    one = jnp.ones((L,), jnp.float32)

    def fill(val):
        def frow(i, _):
            def fcol(j, _):
                ones_v[i, pl.ds(j * L, L)] = val
                return 0

            lax.fori_loop(0, D // L, fcol, 0)
            return 0

        lax.fori_loop(0, CH, frow, 0)

    fill(zero)

    def zacc(k, _):
        pltpu.sync_copy(ones_v, hist_s.at[pl.ds(s * ROWS_PT + k * CH, CH)])
        return 0

    lax.fori_loop(0, ROWS_PT // CH, zacc, 0)
    fill(one)
    plsc.subcore_barrier()

    def body(blk, _):
        base_chunk = wid * CPW + blk * CPB
        pltpu.sync_copy(dst_hbm.at[pl.ds(base_chunk, CPB)], idx_v)
        descs = [
            pltpu.async_copy(ones_v, hist_s.at[idx_v.at[p]], sem, add=True)
            for p in range(CPB)
        ]
        for d in descs:
            d.wait()
        return 0

    lax.fori_loop(0, NB, body, 0)
    plsc.subcore_barrier()

    def wout(k, _):
        base = s * ROWS_PT + k * CH
        pltpu.sync_copy(hist_s.at[pl.ds(base, CH)], ones_v)
        pltpu.sync_copy(ones_v, out_hbm.at[pl.ds(c * NPAD + base, CH)])
        return 0

    lax.fori_loop(0, ROWS_PT // CH, wout, 0)


@functools.partial(
    pl.kernel,
    out_type=jax.ShapeDtypeStruct((NC * NPAD, D), jnp.float32),
    mesh=_mesh,
    scratch_types=[
        pltpu.VMEM((CPB, CH), jnp.int32),
        pltpu.VMEM((CPB, CH), jnp.int32),
        pltpu.VMEM((CH, D), jnp.float32),
        pltpu.VMEM((CH, D), jnp.float32),
        pltpu.VMEM_SHARED((NPAD, D), jnp.float32),
        pltpu.SemaphoreType.DMA,
        pltpu.SemaphoreType.DMA,
    ],
)
def _scatter_kernel(y_hbm, src_hbm, dst_hbm, out_hbm, sidx_v, didx_v,
                    rows0_v, rows1_v, acc_s, sem0, sem1):
    c = lax.axis_index("c")
    s = lax.axis_index("s")
    wid = c * NS + s
    zero = jnp.zeros((L,), jnp.float32)

    def zrow(i, _):
        def zcol(j, _):
            rows0_v[i, pl.ds(j * L, L)] = zero
            return 0

        lax.fori_loop(0, D // L, zcol, 0)
        return 0

    lax.fori_loop(0, CH, zrow, 0)

    def zacc(k, _):
        pltpu.sync_copy(rows0_v, acc_s.at[pl.ds(s * ROWS_PT + k * CH, CH)])
        return 0

    lax.fori_loop(0, ROWS_PT // CH, zacc, 0)
    plsc.subcore_barrier()

    rows = (rows0_v, rows1_v)
    sems = (sem0, sem1)
    base_w = jnp.where(c == 0, s * CPW0, NS * CPW0 + s * CPW1)
    nb = jnp.where(c == 0, NB0, NB1)

    def body(blk, _):
        base_chunk = base_w + blk * CPB
        pltpu.sync_copy(src_hbm.at[pl.ds(base_chunk, CPB)], sidx_v)
        pltpu.sync_copy(dst_hbm.at[pl.ds(base_chunk, CPB)], didx_v)
        descs = [
            pltpu.async_copy(y_hbm.at[sidx_v.at[p]], rows[p % 2],
                             sems[p % 2])
            for p in range(2)
        ]
        for p in range(CPB):
            descs[p].wait()
            pltpu.sync_copy(rows[p % 2], acc_s.at[didx_v.at[p]], add=True)
            if p + 2 < CPB:
                descs.append(
                    pltpu.async_copy(y_hbm.at[sidx_v.at[p + 2]],
                                     rows[p % 2], sems[p % 2]))
        return 0

    lax.fori_loop(0, nb, body, 0)
    plsc.subcore_barrier()

    wdescs = {}
    for k in range(ROWS_PT // CH):
        b = k % 2
        base = s * ROWS_PT + k * CH
        if k >= 2:
            wdescs[k - 2].wait()
        pltpu.sync_copy(acc_s.at[pl.ds(base, CH)], rows[b])
        wdescs[k] = pltpu.async_copy(
            rows[b], out_hbm.at[pl.ds(c * NPAD + base, CH)], sems[b])
    for k in range(ROWS_PT // CH - 2, ROWS_PT // CH):
        wdescs[k].wait()


def _dis_of(h_ref):
    deg = h_ref[0, :, 0] + h_ref[1, :, 0] + 1.0
    return lax.rsqrt(deg)[:, None]


def _tc_first(x_ref, w_ref, h_ref, y_ref):
    xw = jnp.dot(x_ref[...], w_ref[...], preferred_element_type=jnp.float32)
    y_ref[...] = xw * _dis_of(h_ref)


def _tc_mid(acc_ref, y_ref, h_ref, w_ref, b_ref, out_ref):
    dis = _dis_of(h_ref)
    tmp = (acc_ref[0] + acc_ref[1] + y_ref[...]) * dis + b_ref[...]
    hcur = jnp.maximum(tmp, 0.0)
    out_ref[...] = (
        jnp.dot(hcur, w_ref[...], preferred_element_type=jnp.float32) * dis
    )


def _tc_last(acc_ref, y_ref, h_ref, b_ref, out_ref):
    dis = _dis_of(h_ref)
    out_ref[...] = (acc_ref[0] + acc_ref[1] + y_ref[...]) * dis + b_ref[...]


_row_spec = pl.BlockSpec((BM, D), lambda i: (i, 0))
_acc_spec = pl.BlockSpec((NC, BM, D), lambda i: (0, i, 0))
_hist_spec = pl.BlockSpec((NC, BM, D), lambda i: (0, i, 0))
_w_spec = pl.BlockSpec((D, D), lambda i: (0, 0))
_b_spec = pl.BlockSpec((1, D), lambda i: (0, 0))
_out_struct = jax.ShapeDtypeStruct((NPAD, D), jnp.float32)

_tc_first_call = pl.pallas_call(
    _tc_first,
    grid=(NPAD // BM,),
    in_specs=[_row_spec, _w_spec, _hist_spec],
    out_specs=_row_spec,
    out_shape=_out_struct,
)

_tc_mid_call = pl.pallas_call(
    _tc_mid,
    grid=(NPAD // BM,),
    in_specs=[_acc_spec, _row_spec, _hist_spec, _w_spec, _b_spec],
    out_specs=_row_spec,
    out_shape=_out_struct,
)

_tc_last_call = pl.pallas_call(
    _tc_last,
    grid=(NPAD // BM,),
    in_specs=[_acc_spec, _row_spec, _hist_spec, _b_spec],
    out_specs=_row_spec,
    out_shape=_out_struct,
)


def kernel(x, edge_index, W1, b1, W2, b2):
    ei = edge_index.astype(jnp.int32)
    npad_e = E_PAD - E
    src = jnp.concatenate([ei[0], jnp.zeros((npad_e,), jnp.int32)])
    pad_dst = N + jnp.arange(npad_e, dtype=jnp.int32) % (NPAD - N)
    dst = jnp.concatenate([ei[1], pad_dst])
    src2 = src.reshape(E_PAD // CH, CH)
    dst2 = dst.reshape(E_PAD // CH, CH)
    x_p = jnp.pad(x, ((0, NPAD - N), (0, 0)))
    b1r = b1.reshape(1, D)
    b2r = b2.reshape(1, D)

    hist = _deg_kernel(dst2).reshape(NC, NPAD, D)
    y1 = _tc_first_call(x_p, W1, hist)
    acc1 = _scatter_kernel(y1, src2, dst2).reshape(NC, NPAD, D)
    y2 = _tc_mid_call(acc1, y1, hist, W2, b1r)
    acc2 = _scatter_kernel(y2, src2, dst2).reshape(NC, NPAD, D)
    out = _tc_last_call(acc2, y2, hist, b2r)
    return out[:N]
